# baseline (speedup 1.0000x reference)
import jax
import jax.numpy as jnp
from jax import lax
from jax.experimental import pallas as pl
from jax.experimental.pallas import tpu as pltpu
from jax.experimental.pallas import tpu_sc as plsc

NC = 2
NS = 16
NW = NC * NS
C = 128
K0_FRAC = 1 / 3


def _sc_agg_kernel(n_pad, k0, k1, d, interpret=False):
    rps = n_pad // NS
    kmax = max(k0, k1)

    def body(x_hbm, src_hbm, dst_hbm, z_hbm, out_hbm,
             agg_sh, src_v, dst_v, gbuf, sem):
        cid = lax.axis_index("c")
        sid = lax.axis_index("s")
        kw = jnp.where(cid == 0, k0, k1)
        base = jnp.where(cid == 0, sid * k0, NS * k0 + sid * k1)
        base = pl.multiple_of(base, 8)

        pltpu.sync_copy(z_hbm, agg_sh.at[pl.ds(sid * rps, rps)])
        pltpu.sync_copy(src_hbm.at[pl.ds(base, kmax)], src_v)
        pltpu.sync_copy(dst_hbm.at[pl.ds(base, kmax)], dst_v)
        plsc.subcore_barrier()

        def step(j, carry):
            pltpu.async_copy(x_hbm.at[src_v.at[j]], gbuf, sem).wait()
            pltpu.sync_copy(gbuf, agg_sh.at[dst_v.at[j]], add=True)
            return carry

        lax.fori_loop(0, kw, step, 0)
        plsc.subcore_barrier()
        pltpu.sync_copy(agg_sh.at[pl.ds(sid * rps, rps)],
                        out_hbm.at[cid, pl.ds(sid * rps, rps)])

    mesh = plsc.VectorSubcoreMesh(core_axis_name="c", subcore_axis_name="s")
    return pl.kernel(
        body,
        out_type=jax.ShapeDtypeStruct((NC, n_pad, d), jnp.float32),
        mesh=mesh,
        scratch_types=[
            pltpu.VMEM_SHARED((n_pad, d), jnp.float32),
            pltpu.VMEM((kmax, C), jnp.int32),
            pltpu.VMEM((kmax, C), jnp.int32),
            pltpu.VMEM((C, d), jnp.float32),
            pltpu.SemaphoreType.DMA,
        ],
        interpret=interpret,
    )


def _tc_body(p0_ref, p1_ref, x_ref, wt_ref, o_ref):
    agg = p0_ref[...] + p1_ref[...]
    h = jnp.dot(agg, wt_ref[...], preferred_element_type=jnp.float32)
    o_ref[...] = jnp.maximum(h, 0.0) + x_ref[...]


@jax.jit
def kernel(x, edge_index, W):
    n, d = x.shape
    e = edge_index.shape[1]

    kpair = -(-2 * (-(-e // (NW * C))) // 8) * 8
    k0 = max(8, round(kpair * K0_FRAC / 8) * 8)
    k1 = kpair - k0
    kmax = max(k0, k1)
    tot = NS * kpair
    e_pad = tot * C
    n_pad = -(-(n + 1) // (NS * 8)) * (NS * 8)

    src = edge_index[0]
    dst = edge_index[1]
    pad_len = e_pad - e + kmax * C
    pad_dst = n + (jnp.arange(pad_len, dtype=jnp.int32) % (n_pad - n))
    src_p = jnp.concatenate(
        [src, jnp.zeros((pad_len,), jnp.int32)]).reshape(tot + kmax, C)
    dst_p = jnp.concatenate([dst, pad_dst]).reshape(tot + kmax, C)
    zrows = jnp.zeros((n_pad // NS, d), jnp.float32)

    partials = _sc_agg_kernel(n_pad, k0, k1, d)(x, src_p, dst_p, zrows)

    nb = 8 * 125
    out = pl.pallas_call(
        _tc_body,
        out_shape=jax.ShapeDtypeStruct((n, d), jnp.float32),
        grid=(n // nb,),
        in_specs=[
            pl.BlockSpec((nb, d), lambda i: (i, 0)),
            pl.BlockSpec((nb, d), lambda i: (i, 0)),
            pl.BlockSpec((nb, d), lambda i: (i, 0)),
            pl.BlockSpec((d, d), lambda i: (0, 0)),
        ],
        out_specs=pl.BlockSpec((nb, d), lambda i: (i, 0)),
    )(partials[0, :n], partials[1, :n], x, W.T)
    return out


# speedup vs baseline: 1.1664x; 1.1664x over previous
import jax
import jax.numpy as jnp
from jax import lax
from jax.experimental import pallas as pl
from jax.experimental.pallas import tpu as pltpu
from jax.experimental.pallas import tpu_sc as plsc

NC = 2
NS = 16
NW = NC * NS
C = 128
K0_FRAC = 2 / 3


def _sc_agg_kernel(n_pad, k0, k1, d, interpret=False):
    rps = n_pad // NS
    kmax = max(k0, k1)

    def body(x_hbm, src_hbm, dst_hbm, z_hbm, out_hbm,
             agg_sh, src_v, dst_v, gbuf, sem):
        cid = lax.axis_index("c")
        sid = lax.axis_index("s")
        kw = jnp.where(cid == 0, k0, k1)
        base = jnp.where(cid == 0, sid * k0, NS * k0 + sid * k1)
        base = pl.multiple_of(base, 8)

        pltpu.sync_copy(z_hbm, agg_sh.at[pl.ds(sid * rps, rps)])
        pltpu.sync_copy(src_hbm.at[pl.ds(base, kmax)], src_v)
        pltpu.sync_copy(dst_hbm.at[pl.ds(base, kmax)], dst_v)
        plsc.subcore_barrier()

        def step(j, carry):
            pltpu.async_copy(x_hbm.at[src_v.at[j]], gbuf, sem).wait()
            pltpu.sync_copy(gbuf, agg_sh.at[dst_v.at[j]], add=True)
            return carry

        lax.fori_loop(0, kw, step, 0)
        plsc.subcore_barrier()
        pltpu.sync_copy(agg_sh.at[pl.ds(sid * rps, rps)],
                        out_hbm.at[cid, pl.ds(sid * rps, rps)])

    mesh = plsc.VectorSubcoreMesh(core_axis_name="c", subcore_axis_name="s")
    return pl.kernel(
        body,
        out_type=jax.ShapeDtypeStruct((NC, n_pad, d), jnp.float32),
        mesh=mesh,
        scratch_types=[
            pltpu.VMEM_SHARED((n_pad, d), jnp.float32),
            pltpu.VMEM((kmax, C), jnp.int32),
            pltpu.VMEM((kmax, C), jnp.int32),
            pltpu.VMEM((C, d), jnp.float32),
            pltpu.SemaphoreType.DMA,
        ],
        interpret=interpret,
    )


def _tc_body(p0_ref, p1_ref, x_ref, wt_ref, o_ref):
    agg = p0_ref[...] + p1_ref[...]
    h = jnp.dot(agg, wt_ref[...], preferred_element_type=jnp.float32)
    o_ref[...] = jnp.maximum(h, 0.0) + x_ref[...]


@jax.jit
def kernel(x, edge_index, W):
    n, d = x.shape
    e = edge_index.shape[1]

    kpair = -(-2 * (-(-e // (NW * C))) // 8) * 8
    k0 = max(8, round(kpair * K0_FRAC / 8) * 8)
    k1 = kpair - k0
    kmax = max(k0, k1)
    tot = NS * kpair
    e_pad = tot * C
    n_pad = -(-(n + 1) // (NS * 8)) * (NS * 8)

    src = edge_index[0]
    dst = edge_index[1]
    pad_len = e_pad - e + kmax * C
    pad_dst = n + (jnp.arange(pad_len, dtype=jnp.int32) % (n_pad - n))
    src_p = jnp.concatenate(
        [src, jnp.zeros((pad_len,), jnp.int32)]).reshape(tot + kmax, C)
    dst_p = jnp.concatenate([dst, pad_dst]).reshape(tot + kmax, C)
    zrows = jnp.zeros((n_pad // NS, d), jnp.float32)

    partials = _sc_agg_kernel(n_pad, k0, k1, d)(x, src_p, dst_p, zrows)

    nb = 8 * 125
    out = pl.pallas_call(
        _tc_body,
        out_shape=jax.ShapeDtypeStruct((n, d), jnp.float32),
        grid=(n // nb,),
        in_specs=[
            pl.BlockSpec((nb, d), lambda i: (i, 0)),
            pl.BlockSpec((nb, d), lambda i: (i, 0)),
            pl.BlockSpec((nb, d), lambda i: (i, 0)),
            pl.BlockSpec((d, d), lambda i: (0, 0)),
        ],
        out_specs=pl.BlockSpec((nb, d), lambda i: (i, 0)),
    )(partials[0, :n], partials[1, :n], x, W.T)
    return out


# speedup vs baseline: 1.6351x; 1.4018x over previous
import jax
import jax.numpy as jnp
from jax import lax
from jax.experimental import pallas as pl
from jax.experimental.pallas import tpu as pltpu
from jax.experimental.pallas import tpu_sc as plsc

NC = 2
NS = 16
NW = NC * NS
C = 128


def _sc_agg_kernel(n_pad, k, d, interpret=False):
    rps = n_pad // NS

    def body(x_hbm, src_hbm, dst_hbm, z_hbm, out_hbm,
             agg_sh, src_v, dst_v, gbuf, sem):
        cid = lax.axis_index("c")
        sid = lax.axis_index("s")
        wid = sid * NC + cid

        pltpu.sync_copy(z_hbm, agg_sh.at[pl.ds(sid * rps, rps)])
        pltpu.sync_copy(src_hbm.at[wid], src_v)
        pltpu.sync_copy(dst_hbm.at[wid], dst_v)
        plsc.subcore_barrier()

        def step(j, carry):
            pltpu.async_copy(x_hbm.at[src_v.at[j]], gbuf, sem).wait()
            pltpu.sync_copy(gbuf, agg_sh.at[dst_v.at[j]], add=True)
            return carry

        lax.fori_loop(0, k, step, 0)
        plsc.subcore_barrier()
        pltpu.sync_copy(agg_sh.at[pl.ds(sid * rps, rps)],
                        out_hbm.at[cid, pl.ds(sid * rps, rps)])

    mesh = plsc.VectorSubcoreMesh(core_axis_name="c", subcore_axis_name="s")
    return pl.kernel(
        body,
        out_type=jax.ShapeDtypeStruct((NC, n_pad, d), jnp.float32),
        mesh=mesh,
        scratch_types=[
            pltpu.VMEM_SHARED((n_pad, d), jnp.float32),
            pltpu.VMEM((k, C), jnp.int32),
            pltpu.VMEM((k, C), jnp.int32),
            pltpu.VMEM((C, d), jnp.float32),
            pltpu.SemaphoreType.DMA,
        ],
        interpret=interpret,
    )


def _tc_body(p0_ref, p1_ref, x_ref, wt_ref, o_ref):
    agg = p0_ref[...] + p1_ref[...]
    h = jnp.dot(agg, wt_ref[...], preferred_element_type=jnp.float32)
    o_ref[...] = jnp.maximum(h, 0.0) + x_ref[...]


@jax.jit
def kernel(x, edge_index, W):
    n, d = x.shape
    e = edge_index.shape[1]

    k = -(-e // (NW * C))
    e_pad = NW * k * C
    n_pad = -(-(n + 1) // (NS * 8)) * (NS * 8)

    src = edge_index[0]
    dst = edge_index[1]
    pad_dst = n + (jnp.arange(e_pad - e, dtype=jnp.int32) % (n_pad - n))
    src_p = jnp.concatenate(
        [src, jnp.zeros((e_pad - e,), jnp.int32)]).reshape(NW, k, C)
    dst_p = jnp.concatenate([dst, pad_dst]).reshape(NW, k, C)
    zrows = jnp.zeros((n_pad // NS, d), jnp.float32)

    partials = _sc_agg_kernel(n_pad, k, d)(x, src_p, dst_p, zrows)

    nb = 8 * 125
    out = pl.pallas_call(
        _tc_body,
        out_shape=jax.ShapeDtypeStruct((n, d), jnp.float32),
        grid=(n // nb,),
        in_specs=[
            pl.BlockSpec((nb, d), lambda i: (i, 0)),
            pl.BlockSpec((nb, d), lambda i: (i, 0)),
            pl.BlockSpec((nb, d), lambda i: (i, 0)),
            pl.BlockSpec((d, d), lambda i: (0, 0)),
        ],
        out_specs=pl.BlockSpec((nb, d), lambda i: (i, 0)),
    )(partials[0, :n], partials[1, :n], x, W.T)
    return out


# speedup vs baseline: 1.6402x; 1.0031x over previous
import jax
import jax.numpy as jnp
from jax import lax
from jax.experimental import pallas as pl
from jax.experimental.pallas import tpu as pltpu
from jax.experimental.pallas import tpu_sc as plsc

NC = 2
NS = 16
NW = NC * NS
C = 128


def _sc_agg_kernel(n_pad, k, d, interpret=False):
    rps = n_pad // NS

    def body(x_hbm, src_hbm, dst_hbm, z_hbm, out_hbm,
             agg_sh, src_v, dst_v, gbuf, sem, psem):
        cid = lax.axis_index("c")
        sid = lax.axis_index("s")
        wid = sid * NC + cid

        z_cp = pltpu.async_copy(
            z_hbm, agg_sh.at[pl.ds(sid * rps, rps)], psem.at[0])
        s_cp = pltpu.async_copy(src_hbm.at[wid], src_v, psem.at[1])
        d_cp = pltpu.async_copy(dst_hbm.at[wid], dst_v, psem.at[2])
        z_cp.wait()
        s_cp.wait()
        d_cp.wait()
        plsc.subcore_barrier()

        def step(j, carry):
            pltpu.async_copy(x_hbm.at[src_v.at[j]], gbuf, sem).wait()
            pltpu.sync_copy(gbuf, agg_sh.at[dst_v.at[j]], add=True)
            return carry

        lax.fori_loop(0, k, step, 0)
        plsc.subcore_barrier()
        pltpu.sync_copy(agg_sh.at[pl.ds(sid * rps, rps)],
                        out_hbm.at[cid, pl.ds(sid * rps, rps)])

    mesh = plsc.VectorSubcoreMesh(core_axis_name="c", subcore_axis_name="s")
    return pl.kernel(
        body,
        out_type=jax.ShapeDtypeStruct((NC, n_pad, d), jnp.float32),
        mesh=mesh,
        scratch_types=[
            pltpu.VMEM_SHARED((n_pad, d), jnp.float32),
            pltpu.VMEM((k, C), jnp.int32),
            pltpu.VMEM((k, C), jnp.int32),
            pltpu.VMEM((C, d), jnp.float32),
            pltpu.SemaphoreType.DMA,
            pltpu.SemaphoreType.DMA((3,)),
        ],
        interpret=interpret,
    )


def _tc_body(p0_ref, p1_ref, x_ref, wt_ref, o_ref):
    agg = p0_ref[...] + p1_ref[...]
    h = jnp.dot(agg, wt_ref[...], preferred_element_type=jnp.float32)
    o_ref[...] = jnp.maximum(h, 0.0) + x_ref[...]


@jax.jit
def kernel(x, edge_index, W):
    n, d = x.shape
    e = edge_index.shape[1]

    k = -(-e // (NW * C))
    e_pad = NW * k * C
    n_pad = -(-(n + 1) // (NS * 8)) * (NS * 8)

    src = edge_index[0]
    dst = edge_index[1]
    pad_dst = n + (jnp.arange(e_pad - e, dtype=jnp.int32) % (n_pad - n))
    src_p = jnp.concatenate(
        [src, jnp.zeros((e_pad - e,), jnp.int32)]).reshape(NW, k, C)
    dst_p = jnp.concatenate([dst, pad_dst]).reshape(NW, k, C)
    zrows = jnp.zeros((n_pad // NS, d), jnp.float32)

    partials = _sc_agg_kernel(n_pad, k, d)(x, src_p, dst_p, zrows)

    nb = 8 * 125
    out = pl.pallas_call(
        _tc_body,
        out_shape=jax.ShapeDtypeStruct((n, d), jnp.float32),
        grid=(n // nb,),
        in_specs=[
            pl.BlockSpec((nb, d), lambda i: (i, 0)),
            pl.BlockSpec((nb, d), lambda i: (i, 0)),
            pl.BlockSpec((nb, d), lambda i: (i, 0)),
            pl.BlockSpec((d, d), lambda i: (0, 0)),
        ],
        out_specs=pl.BlockSpec((nb, d), lambda i: (i, 0)),
    )(partials[0, :n], partials[1, :n], x, W.T)
    return out
